# Initial kernel scaffold; baseline (speedup 1.0000x reference)
import functools

import jax
import jax.numpy as jnp
from jax import lax
from jax.experimental import pallas as pl
from jax.experimental.pallas import tpu as pltpu
from jax.experimental.pallas import tpu_sc as plsc

N = 10000
F = 128
H = 128
E = 320000

NC = 2
NS = 16
NW = NC * NS
LANES = 16

E_PER_W = E // NW
C = 80
ITERS = E_PER_W // C

ROWS_PER_TILE = N // NS

_MESH = plsc.VectorSubcoreMesh(core_axis_name="c", subcore_axis_name="s")


def _wid(cid, sid):
    return sid * NC + cid


@functools.partial(
    pl.kernel,
    out_type=jax.ShapeDtypeStruct((NC, N), jnp.float32),
    mesh=_MESH,
    scratch_types=[
        pltpu.VMEM((ITERS, C), jnp.int32),
        pltpu.VMEM((C,), jnp.float32),
        pltpu.VMEM_SHARED((N,), jnp.float32),
    ],
)
def _deg_kernel(dst3_hbm, zeros_n_hbm, out_hbm, idx_v, ones_v, deg_sh):
    cid = lax.axis_index("c")
    sid = lax.axis_index("s")
    wid = _wid(cid, sid)

    @pl.when(sid == 0)
    def _():
        pltpu.sync_copy(zeros_n_hbm, deg_sh)

    for k in range(C // LANES):
        ones_v[pl.ds(k * LANES, LANES)] = jnp.ones((LANES,), jnp.float32)
    pltpu.sync_copy(dst3_hbm.at[wid], idx_v)
    plsc.subcore_barrier()

    def body(j, carry):
        pltpu.sync_copy(ones_v, deg_sh.at[idx_v.at[j]], add=True)
        return carry

    lax.fori_loop(0, ITERS, body, 0)
    plsc.subcore_barrier()

    @pl.when(sid == 0)
    def _():
        pltpu.sync_copy(deg_sh, out_hbm.at[cid])


@functools.partial(
    pl.kernel,
    out_type=jax.ShapeDtypeStruct((NC, N, F), jnp.float32),
    mesh=_MESH,
    scratch_types=[
        pltpu.VMEM((ITERS, C), jnp.int32),
        pltpu.VMEM((ITERS, C), jnp.int32),
        pltpu.VMEM((C, F), jnp.float32),
        pltpu.VMEM_SHARED((N, F), jnp.float32),
        pltpu.SemaphoreType.DMA,
    ],
)
def _agg_rows_kernel(src3_hbm, dst3_hbm, u_hbm, zeros_nf_hbm, out_hbm,
                     src_v, dst_v, rows_v, agg_sh, sem):
    cid = lax.axis_index("c")
    sid = lax.axis_index("s")
    wid = _wid(cid, sid)

    r0 = sid * ROWS_PER_TILE
    pltpu.sync_copy(zeros_nf_hbm.at[pl.ds(r0, ROWS_PER_TILE)],
                    agg_sh.at[pl.ds(r0, ROWS_PER_TILE)])
    pltpu.sync_copy(src3_hbm.at[wid], src_v)
    pltpu.sync_copy(dst3_hbm.at[wid], dst_v)
    plsc.subcore_barrier()

    def body(j, carry):
        pltpu.async_copy(u_hbm.at[src_v.at[j]], rows_v, sem).wait()
        pltpu.sync_copy(rows_v, agg_sh.at[dst_v.at[j]], add=True)
        return carry

    lax.fori_loop(0, ITERS, body, 0)
    plsc.subcore_barrier()

    pltpu.sync_copy(agg_sh.at[pl.ds(r0, ROWS_PER_TILE)],
                    out_hbm.at[cid, pl.ds(r0, ROWS_PER_TILE)])


@functools.partial(
    pl.kernel,
    out_type=jax.ShapeDtypeStruct((NC, N), jnp.float32),
    mesh=_MESH,
    scratch_types=[
        pltpu.VMEM((ITERS, C), jnp.int32),
        pltpu.VMEM((ITERS, C), jnp.int32),
        pltpu.VMEM((N,), jnp.float32),
        pltpu.VMEM((C,), jnp.float32),
        pltpu.VMEM_SHARED((N,), jnp.float32),
    ],
)
def _agg_scalar_kernel(src3_hbm, dst3_hbm, uv_hbm, zeros_n_hbm, out_hbm,
                       src_v, dst_v, uv_v, vals_v, acc_sh):
    cid = lax.axis_index("c")
    sid = lax.axis_index("s")
    wid = _wid(cid, sid)

    @pl.when(sid == 0)
    def _():
        pltpu.sync_copy(zeros_n_hbm, acc_sh)

    pltpu.sync_copy(uv_hbm, uv_v)
    pltpu.sync_copy(src3_hbm.at[wid], src_v)
    pltpu.sync_copy(dst3_hbm.at[wid], dst_v)
    plsc.subcore_barrier()

    def body(j, carry):
        for k in range(C // LANES):
            s = src_v[j, pl.ds(k * LANES, LANES)]
            vals_v[pl.ds(k * LANES, LANES)] = plsc.load_gather(uv_v, [s])
        pltpu.sync_copy(vals_v, acc_sh.at[dst_v.at[j]], add=True)
        return carry

    lax.fori_loop(0, ITERS, body, 0)
    plsc.subcore_barrier()

    @pl.when(sid == 0)
    def _():
        pltpu.sync_copy(acc_sh, out_hbm.at[cid])


B = 1000
NB = N // B


def _scale_body(d0_ref, d1_ref, x_ref, dinv_ref, u_ref):
    deg = d0_ref[0, 0, :] + d1_ref[0, 0, :] + 1.0
    dinv = lax.rsqrt(deg)
    dinv_ref[0, 0, :] = dinv
    u_ref[...] = x_ref[...] * dinv[:, None]


def _scale_call(d0, d1, x):
    sspec = pl.BlockSpec((1, 1, B), lambda i: (i, 0, 0))
    fspec = pl.BlockSpec((B, F), lambda i: (i, 0))
    return pl.pallas_call(
        _scale_body,
        grid=(NB,),
        in_specs=[sspec, sspec, fspec],
        out_specs=[sspec, fspec],
        out_shape=[
            jax.ShapeDtypeStruct((NB, 1, B), jnp.float32),
            jax.ShapeDtypeStruct((N, F), jnp.float32),
        ],
    )(d0.reshape(NB, 1, B), d1.reshape(NB, 1, B), x)


def _dense_body(a0_ref, a1_ref, u_ref, dinv_ref, w1_ref, b1_ref, w2_ref,
                uv_ref):
    dinv = dinv_ref[0, 0, :]
    t = (a0_ref[...] + a1_ref[...] + u_ref[...]) * dinv[:, None]
    h = jnp.maximum(
        jnp.dot(t, w1_ref[...], preferred_element_type=jnp.float32)
        + b1_ref[...], 0.0)
    v = jnp.dot(h, w2_ref[...], preferred_element_type=jnp.float32)
    uv_ref[0, 0, :] = dinv * v[:, 0]


def _dense_call(a0, a1, u, dinv3, W1, b1, W2):
    sspec = pl.BlockSpec((1, 1, B), lambda i: (i, 0, 0))
    fspec = pl.BlockSpec((B, F), lambda i: (i, 0))
    return pl.pallas_call(
        _dense_body,
        grid=(NB,),
        in_specs=[
            fspec, fspec, fspec, sspec,
            pl.BlockSpec((F, H), lambda i: (0, 0)),
            pl.BlockSpec((1, H), lambda i: (0, 0)),
            pl.BlockSpec((H, 1), lambda i: (0, 0)),
        ],
        out_specs=sspec,
        out_shape=jax.ShapeDtypeStruct((NB, 1, B), jnp.float32),
    )(a0, a1, u, dinv3, W1, b1.reshape(1, H), W2)


def _final_body(p0_ref, p1_ref, uv_ref, dinv_ref, b2_ref, out_ref):
    out_ref[...] = (dinv_ref[...] * (p0_ref[...] + p1_ref[...] + uv_ref[...])
                    + b2_ref[0, 0])


def _final_call(p0, p1, uv3, dinv3, b2):
    sspec = pl.BlockSpec((1, 1, B), lambda i: (i, 0, 0))
    return pl.pallas_call(
        _final_body,
        grid=(NB,),
        in_specs=[sspec, sspec, sspec, sspec,
                  pl.BlockSpec((1, 1), lambda i: (0, 0))],
        out_specs=sspec,
        out_shape=jax.ShapeDtypeStruct((NB, 1, B), jnp.float32),
    )(p0.reshape(NB, 1, B), p1.reshape(NB, 1, B), uv3, dinv3,
      b2.reshape(1, 1))


def kernel(x, edge_index, W1, b1, W2, b2):
    src = edge_index[0].astype(jnp.int32)
    dst = edge_index[1].astype(jnp.int32)
    src3 = src.reshape(NW, ITERS, C)
    dst3 = dst.reshape(NW, ITERS, C)
    zeros_n = jnp.zeros((N,), jnp.float32)
    zeros_nf = jnp.zeros((N, F), jnp.float32)

    degp = _deg_kernel(dst3, zeros_n)
    dinv3, u = _scale_call(degp[0], degp[1], x)

    aggp = _agg_rows_kernel(src3, dst3, u, zeros_nf)
    uv3 = _dense_call(aggp[0], aggp[1], u, dinv3, W1, b1, W2)

    accp = _agg_scalar_kernel(src3, dst3, uv3.reshape(N), zeros_n)
    out3 = _final_call(accp[0], accp[1], uv3, dinv3, b2)
    return out3.reshape(N)


# speedup vs baseline: 32.0851x; 32.0851x over previous
import functools

import jax
import jax.numpy as jnp
from jax import lax
from jax.experimental import pallas as pl
from jax.experimental.pallas import tpu as pltpu
from jax.experimental.pallas import tpu_sc as plsc

N = 10000
F = 128
H = 128
E = 320000

NC = 2
NS = 16
NW = NC * NS
LANES = 16

E_PER_W = E // NW
C = 80
ITERS = E_PER_W // C

ROWS_PER_TILE = 624
TAIL_ROWS = N - NS * ROWS_PER_TILE
TAIL_R0 = NS * ROWS_PER_TILE

_MESH = plsc.VectorSubcoreMesh(core_axis_name="c", subcore_axis_name="s",
                               num_cores=NC, num_subcores=NS)


def _wid(cid, sid):
    return sid * NC + cid


@functools.partial(
    pl.kernel,
    out_type=jax.ShapeDtypeStruct((NC, N), jnp.float32),
    mesh=_MESH,
    scratch_types=[
        pltpu.VMEM((ITERS, C), jnp.int32),
        pltpu.VMEM((C,), jnp.float32),
        pltpu.VMEM_SHARED((N,), jnp.float32),
    ],
)
def _deg_kernel(dst3_hbm, zeros_n_hbm, out_hbm, idx_v, ones_v, deg_sh):
    cid = lax.axis_index("c")
    sid = lax.axis_index("s")
    wid = _wid(cid, sid)

    @pl.when(sid == 0)
    def _():
        pltpu.sync_copy(zeros_n_hbm, deg_sh)

    for k in range(C // LANES):
        ones_v[pl.ds(k * LANES, LANES)] = jnp.ones((LANES,), jnp.float32)
    pltpu.sync_copy(dst3_hbm.at[wid], idx_v)
    plsc.subcore_barrier()

    def body(j, carry):
        pltpu.sync_copy(ones_v, deg_sh.at[idx_v.at[j]], add=True)
        return carry

    lax.fori_loop(0, ITERS, body, 0)
    plsc.subcore_barrier()

    @pl.when(sid == 0)
    def _():
        pltpu.sync_copy(deg_sh, out_hbm.at[cid])


@functools.partial(
    pl.kernel,
    out_type=jax.ShapeDtypeStruct((NC, N, F), jnp.float32),
    mesh=_MESH,
    scratch_types=[
        pltpu.VMEM((ITERS, C), jnp.int32),
        pltpu.VMEM((ITERS, C), jnp.int32),
        pltpu.VMEM((C, F), jnp.float32),
        pltpu.VMEM_SHARED((N, F), jnp.float32),
        pltpu.SemaphoreType.DMA,
    ],
)
def _agg_rows_kernel(src3_hbm, dst3_hbm, u_hbm, zeros_nf_hbm, out_hbm,
                     src_v, dst_v, rows_v, agg_sh, sem):
    cid = lax.axis_index("c")
    sid = lax.axis_index("s")
    wid = _wid(cid, sid)

    r0 = pl.multiple_of(sid * ROWS_PER_TILE, 8)
    pltpu.sync_copy(zeros_nf_hbm.at[pl.ds(r0, ROWS_PER_TILE)],
                    agg_sh.at[pl.ds(r0, ROWS_PER_TILE)])

    @pl.when(sid == NS - 1)
    def _():
        pltpu.sync_copy(zeros_nf_hbm.at[pl.ds(TAIL_R0, TAIL_ROWS)],
                        agg_sh.at[pl.ds(TAIL_R0, TAIL_ROWS)])

    pltpu.sync_copy(src3_hbm.at[wid], src_v)
    pltpu.sync_copy(dst3_hbm.at[wid], dst_v)
    plsc.subcore_barrier()

    def body(j, carry):
        pltpu.async_copy(u_hbm.at[src_v.at[j]], rows_v, sem).wait()
        pltpu.sync_copy(rows_v, agg_sh.at[dst_v.at[j]], add=True)
        return carry

    lax.fori_loop(0, ITERS, body, 0)
    plsc.subcore_barrier()

    pltpu.sync_copy(agg_sh.at[pl.ds(r0, ROWS_PER_TILE)],
                    out_hbm.at[cid, pl.ds(r0, ROWS_PER_TILE)])

    @pl.when(sid == NS - 1)
    def _():
        pltpu.sync_copy(agg_sh.at[pl.ds(TAIL_R0, TAIL_ROWS)],
                        out_hbm.at[cid, pl.ds(TAIL_R0, TAIL_ROWS)])


@functools.partial(
    pl.kernel,
    out_type=jax.ShapeDtypeStruct((NC, N), jnp.float32),
    mesh=_MESH,
    scratch_types=[
        pltpu.VMEM((ITERS, C), jnp.int32),
        pltpu.VMEM((ITERS, C), jnp.int32),
        pltpu.VMEM((C,), jnp.float32),
        pltpu.VMEM_SHARED((N,), jnp.float32),
        pltpu.VMEM_SHARED((N,), jnp.float32),
        pltpu.SemaphoreType.DMA,
    ],
)
def _agg_scalar_kernel(src3_hbm, dst3_hbm, uv_hbm, zeros_n_hbm, out_hbm,
                       src_v, dst_v, vals_v, uv_sh, acc_sh, sem):
    cid = lax.axis_index("c")
    sid = lax.axis_index("s")
    wid = _wid(cid, sid)

    @pl.when(sid == 0)
    def _():
        pltpu.sync_copy(zeros_n_hbm, acc_sh)
        pltpu.sync_copy(uv_hbm, uv_sh)

    pltpu.sync_copy(src3_hbm.at[wid], src_v)
    pltpu.sync_copy(dst3_hbm.at[wid], dst_v)
    plsc.subcore_barrier()

    def body(j, carry):
        pltpu.async_copy(uv_sh.at[src_v.at[j]], vals_v, sem).wait()
        pltpu.sync_copy(vals_v, acc_sh.at[dst_v.at[j]], add=True)
        return carry

    lax.fori_loop(0, ITERS, body, 0)
    plsc.subcore_barrier()

    @pl.when(sid == 0)
    def _():
        pltpu.sync_copy(acc_sh, out_hbm.at[cid])


B = 1000
NB = N // B


def _scale_body(d0_ref, d1_ref, x_ref, dinv_ref, u_ref):
    deg = d0_ref[0, 0, :] + d1_ref[0, 0, :] + 1.0
    dinv = lax.rsqrt(deg)
    dinv_ref[0, 0, :] = dinv
    u_ref[...] = x_ref[...] * dinv[:, None]


def _scale_call(d0, d1, x):
    sspec = pl.BlockSpec((1, 1, B), lambda i: (i, 0, 0))
    fspec = pl.BlockSpec((B, F), lambda i: (i, 0))
    return pl.pallas_call(
        _scale_body,
        grid=(NB,),
        in_specs=[sspec, sspec, fspec],
        out_specs=[sspec, fspec],
        out_shape=[
            jax.ShapeDtypeStruct((NB, 1, B), jnp.float32),
            jax.ShapeDtypeStruct((N, F), jnp.float32),
        ],
    )(d0.reshape(NB, 1, B), d1.reshape(NB, 1, B), x)


def _dense_body(a0_ref, a1_ref, u_ref, dinv_ref, w1_ref, b1_ref, w2_ref,
                uv_ref):
    dinv = dinv_ref[0, 0, :]
    t = (a0_ref[...] + a1_ref[...] + u_ref[...]) * dinv[:, None]
    h = jnp.maximum(
        jnp.dot(t, w1_ref[...], preferred_element_type=jnp.float32)
        + b1_ref[...], 0.0)
    v = jnp.dot(h, w2_ref[...], preferred_element_type=jnp.float32)
    uv_ref[0, 0, :] = dinv * v[:, 0]


def _dense_call(a0, a1, u, dinv3, W1, b1, W2):
    sspec = pl.BlockSpec((1, 1, B), lambda i: (i, 0, 0))
    fspec = pl.BlockSpec((B, F), lambda i: (i, 0))
    return pl.pallas_call(
        _dense_body,
        grid=(NB,),
        in_specs=[
            fspec, fspec, fspec, sspec,
            pl.BlockSpec((F, H), lambda i: (0, 0)),
            pl.BlockSpec((1, H), lambda i: (0, 0)),
            pl.BlockSpec((H, 1), lambda i: (0, 0)),
        ],
        out_specs=sspec,
        out_shape=jax.ShapeDtypeStruct((NB, 1, B), jnp.float32),
    )(a0, a1, u, dinv3, W1, b1.reshape(1, H), W2)


def _final_body(p0_ref, p1_ref, uv_ref, dinv_ref, b2_ref, out_ref):
    out_ref[...] = (dinv_ref[...] * (p0_ref[...] + p1_ref[...] + uv_ref[...])
                    + b2_ref[0, 0])


def _final_call(p0, p1, uv3, dinv3, b2):
    sspec = pl.BlockSpec((1, 1, B), lambda i: (i, 0, 0))
    return pl.pallas_call(
        _final_body,
        grid=(NB,),
        in_specs=[sspec, sspec, sspec, sspec,
                  pl.BlockSpec((1, 1), lambda i: (0, 0))],
        out_specs=sspec,
        out_shape=jax.ShapeDtypeStruct((NB, 1, B), jnp.float32),
    )(p0.reshape(NB, 1, B), p1.reshape(NB, 1, B), uv3, dinv3,
      b2.reshape(1, 1))


def kernel(x, edge_index, W1, b1, W2, b2):
    src = edge_index[0].astype(jnp.int32)
    dst = edge_index[1].astype(jnp.int32)
    src3 = src.reshape(NW, ITERS, C)
    dst3 = dst.reshape(NW, ITERS, C)
    zeros_n = jnp.zeros((N,), jnp.float32)
    zeros_nf = jnp.zeros((N, F), jnp.float32)

    degp = _deg_kernel(dst3, zeros_n)
    dinv3, u = _scale_call(degp[0], degp[1], x)

    aggp = _agg_rows_kernel(src3, dst3, u, zeros_nf)
    uv3 = _dense_call(aggp[0], aggp[1], u, dinv3, W1, b1, W2)

    accp = _agg_scalar_kernel(src3, dst3, uv3.reshape(N), zeros_n)
    out3 = _final_call(accp[0], accp[1], uv3, dinv3, b2)
    return out3.reshape(N)


# speedup vs baseline: 49.2180x; 1.5340x over previous
import functools

import jax
import jax.numpy as jnp
from jax import lax
from jax.experimental import pallas as pl
from jax.experimental.pallas import tpu as pltpu
from jax.experimental.pallas import tpu_sc as plsc

N = 10000
F = 128
H = 128
E = 320000

NC = 2
NS = 16
NW = NC * NS
LANES = 16

E_PER_W = E // NW
C = 100
ITERS = E_PER_W // C
NBUF = 2
WAVE = 20

ROWS_PER_TILE = 624
TAIL_ROWS = N - NS * ROWS_PER_TILE
TAIL_R0 = NS * ROWS_PER_TILE

_MESH = plsc.VectorSubcoreMesh(core_axis_name="c", subcore_axis_name="s",
                               num_cores=NC, num_subcores=NS)

_SC_PARAMS = pltpu.CompilerParams(use_tc_tiling_on_sc=False)


def _wid(cid, sid):
    return sid * NC + cid


DEG_C = 125
DEG_ITERS = E_PER_W // DEG_C
DEG_WAVES = DEG_ITERS // WAVE


@functools.partial(
    pl.kernel,
    out_type=jax.ShapeDtypeStruct((NC, N), jnp.float32),
    mesh=_MESH,
    compiler_params=_SC_PARAMS,
    scratch_types=[
        pltpu.VMEM((DEG_ITERS, DEG_C), jnp.int32),
        pltpu.VMEM((128,), jnp.float32),
        pltpu.VMEM_SHARED((N,), jnp.float32),
        pltpu.SemaphoreType.DMA,
    ],
)
def _deg_kernel(dstd_hbm, zeros_n_hbm, out_hbm, idx_v, ones_v, deg_sh, sem):
    cid = lax.axis_index("c")
    sid = lax.axis_index("s")
    wid = _wid(cid, sid)

    @pl.when(sid == 0)
    def _():
        pltpu.sync_copy(zeros_n_hbm, deg_sh)

    for k in range(128 // LANES):
        ones_v[pl.ds(k * LANES, LANES)] = jnp.ones((LANES,), jnp.float32)
    pltpu.sync_copy(dstd_hbm.at[wid], idx_v)
    plsc.subcore_barrier()

    ones_c = ones_v.at[pl.ds(0, DEG_C)]

    def wave_body(w, carry):
        for k in range(WAVE):
            j = w * WAVE + k
            pltpu.async_copy(ones_c, deg_sh.at[idx_v.at[j]], sem, add=True)
        for k in range(WAVE):
            pltpu.make_async_copy(
                ones_c, deg_sh.at[idx_v.at[w * WAVE + k]], sem).wait()
        return carry

    lax.fori_loop(0, DEG_WAVES, wave_body, 0)
    plsc.subcore_barrier()

    @pl.when(sid == 0)
    def _():
        pltpu.sync_copy(deg_sh, out_hbm.at[cid])


@functools.partial(
    pl.kernel,
    out_type=jax.ShapeDtypeStruct((NC, N, F), jnp.float32),
    mesh=_MESH,
    compiler_params=_SC_PARAMS,
    scratch_types=(
        [pltpu.VMEM((ITERS, C), jnp.int32),
         pltpu.VMEM((ITERS, C), jnp.int32),
         pltpu.VMEM_SHARED((N, F), jnp.float32)]
        + [pltpu.VMEM((C, F), jnp.float32) for _ in range(NBUF)]
        + [pltpu.SemaphoreType.DMA for _ in range(2 * NBUF)]
    ),
)
def _agg_rows_kernel(src3_hbm, dst3_hbm, u_hbm, zeros_nf_hbm, out_hbm,
                     src_v, dst_v, agg_sh, *bufs_and_sems):
    rows = bufs_and_sems[:NBUF]
    gsem = bufs_and_sems[NBUF:2 * NBUF]
    ssem = bufs_and_sems[2 * NBUF:]
    cid = lax.axis_index("c")
    sid = lax.axis_index("s")
    wid = _wid(cid, sid)

    r0 = pl.multiple_of(sid * ROWS_PER_TILE, 8)
    pltpu.sync_copy(zeros_nf_hbm.at[pl.ds(r0, ROWS_PER_TILE)],
                    agg_sh.at[pl.ds(r0, ROWS_PER_TILE)])

    @pl.when(sid == NS - 1)
    def _():
        pltpu.sync_copy(zeros_nf_hbm.at[pl.ds(TAIL_R0, TAIL_ROWS)],
                        agg_sh.at[pl.ds(TAIL_R0, TAIL_ROWS)])

    pltpu.sync_copy(src3_hbm.at[wid], src_v)
    pltpu.sync_copy(dst3_hbm.at[wid], dst_v)
    plsc.subcore_barrier()

    for b in range(NBUF):
        pltpu.async_copy(u_hbm.at[src_v.at[b]], rows[b], gsem[b])

    def round_body(jj, carry):
        for b in range(NBUF):
            j = jj * NBUF + b
            pltpu.make_async_copy(u_hbm.at[src_v.at[j]], rows[b],
                                  gsem[b]).wait()
            pltpu.async_copy(rows[b], agg_sh.at[dst_v.at[j]], ssem[b],
                             add=True)
            jn = j + NBUF

            @pl.when(jn < ITERS)
            def _():
                pltpu.make_async_copy(rows[b], agg_sh.at[dst_v.at[j]],
                                      ssem[b]).wait()
                pltpu.async_copy(u_hbm.at[src_v.at[jn]], rows[b], gsem[b])
        return carry

    lax.fori_loop(0, ITERS // NBUF, round_body, 0)
    for b in range(ITERS % NBUF):
        j = (ITERS // NBUF) * NBUF + b
        pltpu.make_async_copy(u_hbm.at[src_v.at[j]], rows[b], gsem[b]).wait()
        pltpu.async_copy(rows[b], agg_sh.at[dst_v.at[j]], ssem[b], add=True)
    for t in range(NBUF):
        j = ITERS - NBUF + t
        pltpu.make_async_copy(rows[j % NBUF], agg_sh.at[dst_v.at[j]],
                              ssem[j % NBUF]).wait()
    plsc.subcore_barrier()

    pltpu.sync_copy(agg_sh.at[pl.ds(r0, ROWS_PER_TILE)],
                    out_hbm.at[cid, pl.ds(r0, ROWS_PER_TILE)])

    @pl.when(sid == NS - 1)
    def _():
        pltpu.sync_copy(agg_sh.at[pl.ds(TAIL_R0, TAIL_ROWS)],
                        out_hbm.at[cid, pl.ds(TAIL_R0, TAIL_ROWS)])


@functools.partial(
    pl.kernel,
    out_type=jax.ShapeDtypeStruct((NC, N), jnp.float32),
    mesh=_MESH,
    compiler_params=_SC_PARAMS,
    scratch_types=[
        pltpu.VMEM((ITERS, C), jnp.int32),
        pltpu.VMEM((ITERS, C), jnp.int32),
        pltpu.VMEM((ITERS, C), jnp.float32),
        pltpu.VMEM_SHARED((N,), jnp.float32),
        pltpu.VMEM_SHARED((N,), jnp.float32),
        pltpu.SemaphoreType.DMA,
        pltpu.SemaphoreType.DMA,
    ],
)
def _agg_scalar_kernel(src3_hbm, dst3_hbm, uv_hbm, zeros_n_hbm, out_hbm,
                       src_v, dst_v, vals_v, uv_sh, acc_sh, gsem, ssem):
    cid = lax.axis_index("c")
    sid = lax.axis_index("s")
    wid = _wid(cid, sid)

    @pl.when(sid == 0)
    def _():
        pltpu.sync_copy(zeros_n_hbm, acc_sh)
        pltpu.sync_copy(uv_hbm, uv_sh)

    pltpu.sync_copy(src3_hbm.at[wid], src_v)
    pltpu.sync_copy(dst3_hbm.at[wid], dst_v)
    plsc.subcore_barrier()

    def gfire(j, carry):
        pltpu.async_copy(uv_sh.at[src_v.at[j]], vals_v.at[j], gsem)
        return carry

    def gdrain_sfire(j, carry):
        pltpu.make_async_copy(uv_sh.at[src_v.at[j]], vals_v.at[j],
                              gsem).wait()
        pltpu.async_copy(vals_v.at[j], acc_sh.at[dst_v.at[j]], ssem,
                         add=True)
        return carry

    def sdrain(j, carry):
        pltpu.make_async_copy(vals_v.at[j], acc_sh.at[dst_v.at[j]],
                              ssem).wait()
        return carry

    lax.fori_loop(0, ITERS, gfire, 0)
    lax.fori_loop(0, ITERS, gdrain_sfire, 0)
    lax.fori_loop(0, ITERS, sdrain, 0)
    plsc.subcore_barrier()

    @pl.when(sid == 0)
    def _():
        pltpu.sync_copy(acc_sh, out_hbm.at[cid])


B = 1000
NB = N // B


def _matmul_body(x_ref, w1_ref, xw_ref):
    xw_ref[...] = jnp.dot(x_ref[...], w1_ref[...],
                          preferred_element_type=jnp.float32)


def _matmul_call(x, W1):
    fspec = pl.BlockSpec((B, F), lambda i: (i, 0))
    return pl.pallas_call(
        _matmul_body,
        grid=(NB,),
        in_specs=[fspec, pl.BlockSpec((F, H), lambda i: (0, 0))],
        out_specs=pl.BlockSpec((B, H), lambda i: (i, 0)),
        out_shape=jax.ShapeDtypeStruct((N, H), jnp.float32),
    )(x, W1)


def _scale_body(d0_ref, d1_ref, xw_ref, dinv_ref, u_ref):
    deg = d0_ref[0, 0, :] + d1_ref[0, 0, :] + 1.0
    dinv = lax.rsqrt(deg)
    dinv_ref[0, 0, :] = dinv
    u_ref[...] = xw_ref[...] * dinv[:, None]


def _scale_call(d0, d1, xw):
    sspec = pl.BlockSpec((1, 1, B), lambda i: (i, 0, 0))
    fspec = pl.BlockSpec((B, H), lambda i: (i, 0))
    return pl.pallas_call(
        _scale_body,
        grid=(NB,),
        in_specs=[sspec, sspec, fspec],
        out_specs=[sspec, fspec],
        out_shape=[
            jax.ShapeDtypeStruct((NB, 1, B), jnp.float32),
            jax.ShapeDtypeStruct((N, H), jnp.float32),
        ],
    )(d0.reshape(NB, 1, B), d1.reshape(NB, 1, B), xw)


def _dense_body(a0_ref, a1_ref, u_ref, dinv_ref, b1_ref, w2_ref, uv_ref):
    dinv = dinv_ref[0, 0, :]
    h = jnp.maximum(
        (a0_ref[...] + a1_ref[...] + u_ref[...]) * dinv[:, None]
        + b1_ref[...], 0.0)
    v = jnp.dot(h, w2_ref[...], preferred_element_type=jnp.float32)
    uv_ref[0, 0, :] = dinv * v[:, 0]


def _dense_call(a0, a1, u, dinv3, b1, W2):
    sspec = pl.BlockSpec((1, 1, B), lambda i: (i, 0, 0))
    fspec = pl.BlockSpec((B, H), lambda i: (i, 0))
    return pl.pallas_call(
        _dense_body,
        grid=(NB,),
        in_specs=[
            fspec, fspec, fspec, sspec,
            pl.BlockSpec((1, H), lambda i: (0, 0)),
            pl.BlockSpec((H, 1), lambda i: (0, 0)),
        ],
        out_specs=sspec,
        out_shape=jax.ShapeDtypeStruct((NB, 1, B), jnp.float32),
    )(a0, a1, u, dinv3, b1.reshape(1, H), W2)


def _final_body(p0_ref, p1_ref, uv_ref, dinv_ref, b2_ref, out_ref):
    out_ref[...] = (dinv_ref[...] * (p0_ref[...] + p1_ref[...] + uv_ref[...])
                    + b2_ref[0, 0])


def _final_call(p0, p1, uv3, dinv3, b2):
    sspec = pl.BlockSpec((1, 1, B), lambda i: (i, 0, 0))
    return pl.pallas_call(
        _final_body,
        grid=(NB,),
        in_specs=[sspec, sspec, sspec, sspec,
                  pl.BlockSpec((1, 1), lambda i: (0, 0))],
        out_specs=sspec,
        out_shape=jax.ShapeDtypeStruct((NB, 1, B), jnp.float32),
    )(p0.reshape(NB, 1, B), p1.reshape(NB, 1, B), uv3, dinv3,
      b2.reshape(1, 1))


def kernel(x, edge_index, W1, b1, W2, b2):
    src = edge_index[0].astype(jnp.int32)
    dst = edge_index[1].astype(jnp.int32)
    src3 = src.reshape(NW, ITERS, C)
    dst3 = dst.reshape(NW, ITERS, C)
    dstd = dst.reshape(NW, DEG_ITERS, DEG_C)
    zeros_n = jnp.zeros((N,), jnp.float32)
    zeros_nf = jnp.zeros((N, F), jnp.float32)

    xw = _matmul_call(x, W1)
    degp = _deg_kernel(dstd, zeros_n)
    dinv3, u = _scale_call(degp[0], degp[1], xw)

    aggp = _agg_rows_kernel(src3, dst3, u, zeros_nf)
    uv3 = _dense_call(aggp[0], aggp[1], u, dinv3, b1, W2)

    accp = _agg_scalar_kernel(src3, dst3, uv3.reshape(N), zeros_n)
    out3 = _final_call(accp[0], accp[1], uv3, dinv3, b2)
    return out3.reshape(N)


# speedup vs baseline: 51.3304x; 1.0429x over previous
import functools

import jax
import jax.numpy as jnp
from jax import lax
from jax.experimental import pallas as pl
from jax.experimental.pallas import tpu as pltpu
from jax.experimental.pallas import tpu_sc as plsc

N = 10000
F = 128
H = 128
E = 320000

NC = 2
NS = 16
NW = NC * NS
LANES = 16

E_PER_W = E // NW
C = 100
ITERS = E_PER_W // C
NBUF = 2
WAVE = 20

ROWS_PER_TILE = 624
TAIL_ROWS = N - NS * ROWS_PER_TILE
TAIL_R0 = NS * ROWS_PER_TILE

_MESH = plsc.VectorSubcoreMesh(core_axis_name="c", subcore_axis_name="s",
                               num_cores=NC, num_subcores=NS)

_SC_PARAMS = pltpu.CompilerParams(use_tc_tiling_on_sc=False)


def _wid(cid, sid):
    return sid * NC + cid


DEG_C = 125
DEG_ITERS = E_PER_W // DEG_C
DEG_WAVES = DEG_ITERS // WAVE


@functools.partial(
    pl.kernel,
    out_type=jax.ShapeDtypeStruct((NC, N), jnp.float32),
    mesh=_MESH,
    compiler_params=_SC_PARAMS,
    scratch_types=[
        pltpu.VMEM((DEG_ITERS, DEG_C), jnp.int32),
        pltpu.VMEM((128,), jnp.float32),
        pltpu.VMEM_SHARED((N,), jnp.float32),
        pltpu.SemaphoreType.DMA,
    ],
)
def _deg_kernel(dstd_hbm, zeros_n_hbm, out_hbm, idx_v, ones_v, deg_sh, sem):
    cid = lax.axis_index("c")
    sid = lax.axis_index("s")
    wid = _wid(cid, sid)

    @pl.when(sid == 0)
    def _():
        pltpu.sync_copy(zeros_n_hbm, deg_sh)

    for k in range(128 // LANES):
        ones_v[pl.ds(k * LANES, LANES)] = jnp.ones((LANES,), jnp.float32)
    pltpu.sync_copy(dstd_hbm.at[wid], idx_v)
    plsc.subcore_barrier()

    ones_c = ones_v.at[pl.ds(0, DEG_C)]

    def wave_body(w, carry):
        for k in range(WAVE):
            j = w * WAVE + k
            pltpu.async_copy(ones_c, deg_sh.at[idx_v.at[j]], sem, add=True)
        for k in range(WAVE):
            pltpu.make_async_copy(
                ones_c, deg_sh.at[idx_v.at[w * WAVE + k]], sem).wait()
        return carry

    lax.fori_loop(0, DEG_WAVES, wave_body, 0)
    plsc.subcore_barrier()

    @pl.when(sid == 0)
    def _():
        pltpu.sync_copy(deg_sh, out_hbm.at[cid])


@functools.partial(
    pl.kernel,
    out_type=jax.ShapeDtypeStruct((NC, N, F), jnp.float32),
    mesh=_MESH,
    compiler_params=_SC_PARAMS,
    scratch_types=(
        [pltpu.VMEM((ITERS, C), jnp.int32),
         pltpu.VMEM((ITERS, C), jnp.int32),
         pltpu.VMEM_SHARED((N, F), jnp.float32)]
        + [pltpu.VMEM((C, F), jnp.float32) for _ in range(NBUF)]
        + [pltpu.SemaphoreType.DMA for _ in range(2 * NBUF)]
    ),
)
def _agg_rows_kernel(src3_hbm, dst3_hbm, u_hbm, zeros_nf_hbm, out_hbm,
                     src_v, dst_v, agg_sh, *bufs_and_sems):
    rows = bufs_and_sems[:NBUF]
    gsem = bufs_and_sems[NBUF:2 * NBUF]
    ssem = bufs_and_sems[2 * NBUF:]
    cid = lax.axis_index("c")
    sid = lax.axis_index("s")
    wid = _wid(cid, sid)

    r0 = pl.multiple_of(sid * ROWS_PER_TILE, 8)
    @pl.when(cid == 0)
    def _():
        pltpu.sync_copy(u_hbm.at[pl.ds(r0, ROWS_PER_TILE)],
                        agg_sh.at[pl.ds(r0, ROWS_PER_TILE)])

        @pl.when(sid == NS - 1)
        def _():
            pltpu.sync_copy(u_hbm.at[pl.ds(TAIL_R0, TAIL_ROWS)],
                            agg_sh.at[pl.ds(TAIL_R0, TAIL_ROWS)])

    @pl.when(cid == 1)
    def _():
        pltpu.sync_copy(zeros_nf_hbm.at[pl.ds(r0, ROWS_PER_TILE)],
                        agg_sh.at[pl.ds(r0, ROWS_PER_TILE)])

        @pl.when(sid == NS - 1)
        def _():
            pltpu.sync_copy(zeros_nf_hbm.at[pl.ds(TAIL_R0, TAIL_ROWS)],
                            agg_sh.at[pl.ds(TAIL_R0, TAIL_ROWS)])

    pltpu.sync_copy(src3_hbm.at[wid], src_v)
    pltpu.sync_copy(dst3_hbm.at[wid], dst_v)
    plsc.subcore_barrier()

    for b in range(NBUF):
        pltpu.async_copy(u_hbm.at[src_v.at[b]], rows[b], gsem[b])

    def round_body(jj, carry):
        for b in range(NBUF):
            j = jj * NBUF + b
            pltpu.make_async_copy(u_hbm.at[src_v.at[j]], rows[b],
                                  gsem[b]).wait()
            pltpu.async_copy(rows[b], agg_sh.at[dst_v.at[j]], ssem[b],
                             add=True)
            jn = j + NBUF

            @pl.when(jn < ITERS)
            def _():
                pltpu.make_async_copy(rows[b], agg_sh.at[dst_v.at[j]],
                                      ssem[b]).wait()
                pltpu.async_copy(u_hbm.at[src_v.at[jn]], rows[b], gsem[b])
        return carry

    lax.fori_loop(0, ITERS // NBUF, round_body, 0)
    for b in range(ITERS % NBUF):
        j = (ITERS // NBUF) * NBUF + b
        pltpu.make_async_copy(u_hbm.at[src_v.at[j]], rows[b], gsem[b]).wait()
        pltpu.async_copy(rows[b], agg_sh.at[dst_v.at[j]], ssem[b], add=True)
    for t in range(NBUF):
        j = ITERS - NBUF + t
        pltpu.make_async_copy(rows[j % NBUF], agg_sh.at[dst_v.at[j]],
                              ssem[j % NBUF]).wait()
    plsc.subcore_barrier()

    pltpu.sync_copy(agg_sh.at[pl.ds(r0, ROWS_PER_TILE)],
                    out_hbm.at[cid, pl.ds(r0, ROWS_PER_TILE)])

    @pl.when(sid == NS - 1)
    def _():
        pltpu.sync_copy(agg_sh.at[pl.ds(TAIL_R0, TAIL_ROWS)],
                        out_hbm.at[cid, pl.ds(TAIL_R0, TAIL_ROWS)])


@functools.partial(
    pl.kernel,
    out_type=jax.ShapeDtypeStruct((NC, N), jnp.float32),
    mesh=_MESH,
    compiler_params=_SC_PARAMS,
    scratch_types=[
        pltpu.VMEM((ITERS, C), jnp.int32),
        pltpu.VMEM((ITERS, C), jnp.int32),
        pltpu.VMEM((ITERS, C), jnp.float32),
        pltpu.VMEM_SHARED((N,), jnp.float32),
        pltpu.VMEM_SHARED((N,), jnp.float32),
        pltpu.SemaphoreType.DMA,
        pltpu.SemaphoreType.DMA,
    ],
)
def _agg_scalar_kernel(src3_hbm, dst3_hbm, uv_hbm, zeros_n_hbm, out_hbm,
                       src_v, dst_v, vals_v, uv_sh, acc_sh, gsem, ssem):
    cid = lax.axis_index("c")
    sid = lax.axis_index("s")
    wid = _wid(cid, sid)

    @pl.when(sid == 0)
    def _():
        pltpu.sync_copy(zeros_n_hbm, acc_sh)
        pltpu.sync_copy(uv_hbm, uv_sh)

    pltpu.sync_copy(src3_hbm.at[wid], src_v)
    pltpu.sync_copy(dst3_hbm.at[wid], dst_v)
    plsc.subcore_barrier()

    def gfire(j, carry):
        pltpu.async_copy(uv_sh.at[src_v.at[j]], vals_v.at[j], gsem)
        return carry

    def gdrain_sfire(j, carry):
        pltpu.make_async_copy(uv_sh.at[src_v.at[j]], vals_v.at[j],
                              gsem).wait()
        pltpu.async_copy(vals_v.at[j], acc_sh.at[dst_v.at[j]], ssem,
                         add=True)
        return carry

    def sdrain(j, carry):
        pltpu.make_async_copy(vals_v.at[j], acc_sh.at[dst_v.at[j]],
                              ssem).wait()
        return carry

    lax.fori_loop(0, ITERS, gfire, 0)
    lax.fori_loop(0, ITERS, gdrain_sfire, 0)
    lax.fori_loop(0, ITERS, sdrain, 0)
    plsc.subcore_barrier()

    @pl.when(sid == 0)
    def _():
        pltpu.sync_copy(acc_sh, out_hbm.at[cid])


B = 1000
NB = N // B


def _scale_body(d0_ref, d1_ref, x_ref, w1_ref, dinv_ref, u_ref):
    deg = d0_ref[0, 0, :] + d1_ref[0, 0, :] + 1.0
    dinv = lax.rsqrt(deg)
    dinv_ref[0, 0, :] = dinv
    xw = jnp.dot(x_ref[...], w1_ref[...], preferred_element_type=jnp.float32)
    u_ref[...] = xw * dinv[:, None]


def _scale_call(d0, d1, x, W1):
    sspec = pl.BlockSpec((1, 1, B), lambda i: (i, 0, 0))
    fspec = pl.BlockSpec((B, F), lambda i: (i, 0))
    return pl.pallas_call(
        _scale_body,
        grid=(NB,),
        in_specs=[sspec, sspec, fspec,
                  pl.BlockSpec((F, H), lambda i: (0, 0))],
        out_specs=[sspec, pl.BlockSpec((B, H), lambda i: (i, 0))],
        out_shape=[
            jax.ShapeDtypeStruct((NB, 1, B), jnp.float32),
            jax.ShapeDtypeStruct((N, H), jnp.float32),
        ],
    )(d0.reshape(NB, 1, B), d1.reshape(NB, 1, B), x, W1)


def _dense_body(a0_ref, a1_ref, dinv_ref, b1_ref, w2_ref, uv_ref):
    dinv = dinv_ref[0, 0, :]
    s = a0_ref[...].astype(jnp.float32) + a1_ref[...].astype(jnp.float32)
    h = jnp.maximum(s * dinv[:, None] + b1_ref[...], 0.0)
    v = jnp.dot(h, w2_ref[...], preferred_element_type=jnp.float32)
    uv_ref[0, 0, :] = dinv * v[:, 0]


def _dense_call(a0, a1, dinv3, b1, W2):
    sspec = pl.BlockSpec((1, 1, B), lambda i: (i, 0, 0))
    fspec = pl.BlockSpec((B, H), lambda i: (i, 0))
    return pl.pallas_call(
        _dense_body,
        grid=(NB,),
        in_specs=[
            fspec, fspec, sspec,
            pl.BlockSpec((1, H), lambda i: (0, 0)),
            pl.BlockSpec((H, 1), lambda i: (0, 0)),
        ],
        out_specs=sspec,
        out_shape=jax.ShapeDtypeStruct((NB, 1, B), jnp.float32),
    )(a0, a1, dinv3, b1.reshape(1, H), W2)


def _final_body(p0_ref, p1_ref, uv_ref, dinv_ref, b2_ref, out_ref):
    out_ref[...] = (dinv_ref[...] * (p0_ref[...] + p1_ref[...] + uv_ref[...])
                    + b2_ref[0, 0])


def _final_call(p0, p1, uv3, dinv3, b2):
    sspec = pl.BlockSpec((1, 1, B), lambda i: (i, 0, 0))
    return pl.pallas_call(
        _final_body,
        grid=(NB,),
        in_specs=[sspec, sspec, sspec, sspec,
                  pl.BlockSpec((1, 1), lambda i: (0, 0))],
        out_specs=sspec,
        out_shape=jax.ShapeDtypeStruct((NB, 1, B), jnp.float32),
    )(p0.reshape(NB, 1, B), p1.reshape(NB, 1, B), uv3, dinv3,
      b2.reshape(1, 1))


def kernel(x, edge_index, W1, b1, W2, b2):
    src = edge_index[0].astype(jnp.int32)
    dst = edge_index[1].astype(jnp.int32)
    src3 = src.reshape(NW, ITERS, C)
    dst3 = dst.reshape(NW, ITERS, C)
    dstd = dst.reshape(NW, DEG_ITERS, DEG_C)
    zeros_n = jnp.zeros((N,), jnp.float32)
    zeros_nf = jnp.zeros((N, F), jnp.float32)

    degp = _deg_kernel(dstd, zeros_n)
    dinv3, u = _scale_call(degp[0], degp[1], x, W1)

    aggp = _agg_rows_kernel(src3, dst3, u, zeros_nf)
    uv3 = _dense_call(aggp[0], aggp[1], dinv3, b1, W2)

    accp = _agg_scalar_kernel(src3, dst3, uv3.reshape(N), zeros_n)
    out3 = _final_call(accp[0], accp[1], uv3, dinv3, b2)
    return out3.reshape(N)


# speedup vs baseline: 51.4362x; 1.0021x over previous
import functools

import jax
import jax.numpy as jnp
from jax import lax
from jax.experimental import pallas as pl
from jax.experimental.pallas import tpu as pltpu
from jax.experimental.pallas import tpu_sc as plsc

N = 10000
F = 128
H = 128
E = 320000

NC = 2
NS = 16
NW = NC * NS
LANES = 16

E_PER_W = E // NW
C = 100
ITERS = E_PER_W // C
NBUF = 2
WAVE = 20

ROWS_PER_TILE = 624
TAIL_ROWS = N - NS * ROWS_PER_TILE
TAIL_R0 = NS * ROWS_PER_TILE

_MESH = plsc.VectorSubcoreMesh(core_axis_name="c", subcore_axis_name="s",
                               num_cores=NC, num_subcores=NS)

_SC_PARAMS = pltpu.CompilerParams(use_tc_tiling_on_sc=False)


def _wid(cid, sid):
    return sid * NC + cid


DEG_C = 125
DEG_ITERS = E_PER_W // DEG_C
DEG_WAVES = DEG_ITERS // WAVE


@functools.partial(
    pl.kernel,
    out_type=jax.ShapeDtypeStruct((NC, N), jnp.float32),
    mesh=_MESH,
    compiler_params=_SC_PARAMS,
    scratch_types=[
        pltpu.VMEM((DEG_ITERS, DEG_C), jnp.int32),
        pltpu.VMEM((128,), jnp.float32),
        pltpu.VMEM_SHARED((N,), jnp.float32),
        pltpu.SemaphoreType.DMA,
    ],
)
def _deg_kernel(dstd_hbm, zeros_n_hbm, out_hbm, idx_v, ones_v, deg_sh, sem):
    cid = lax.axis_index("c")
    sid = lax.axis_index("s")
    wid = _wid(cid, sid)

    @pl.when(sid == 0)
    def _():
        pltpu.sync_copy(zeros_n_hbm, deg_sh)

    for k in range(128 // LANES):
        ones_v[pl.ds(k * LANES, LANES)] = jnp.ones((LANES,), jnp.float32)
    pltpu.sync_copy(dstd_hbm.at[wid], idx_v)
    plsc.subcore_barrier()

    ones_c = ones_v.at[pl.ds(0, DEG_C)]

    def wave_body(w, carry):
        for k in range(WAVE):
            j = w * WAVE + k
            pltpu.async_copy(ones_c, deg_sh.at[idx_v.at[j]], sem, add=True)
        for k in range(WAVE):
            pltpu.make_async_copy(
                ones_c, deg_sh.at[idx_v.at[w * WAVE + k]], sem).wait()
        return carry

    lax.fori_loop(0, DEG_WAVES, wave_body, 0)
    plsc.subcore_barrier()

    @pl.when(sid == 0)
    def _():
        pltpu.sync_copy(deg_sh, out_hbm.at[cid])


@functools.partial(
    pl.kernel,
    out_type=jax.ShapeDtypeStruct((NC, N, F), jnp.float32),
    mesh=_MESH,
    compiler_params=_SC_PARAMS,
    scratch_types=(
        [pltpu.VMEM((ITERS, C), jnp.int32),
         pltpu.VMEM((ITERS, C), jnp.int32),
         pltpu.VMEM_SHARED((N, F), jnp.float32)]
        + [pltpu.VMEM((C, F), jnp.float32) for _ in range(NBUF)]
        + [pltpu.SemaphoreType.DMA for _ in range(2 * NBUF)]
    ),
)
def _agg_rows_kernel(src3_hbm, dst3_hbm, u_hbm, zeros_nf_hbm, out_hbm,
                     src_v, dst_v, agg_sh, *bufs_and_sems):
    rows = bufs_and_sems[:NBUF]
    gsem = bufs_and_sems[NBUF:2 * NBUF]
    ssem = bufs_and_sems[2 * NBUF:]
    cid = lax.axis_index("c")
    sid = lax.axis_index("s")
    wid = _wid(cid, sid)

    r0 = pl.multiple_of(sid * ROWS_PER_TILE, 8)
    @pl.when(cid == 0)
    def _():
        pltpu.sync_copy(u_hbm.at[pl.ds(r0, ROWS_PER_TILE)],
                        agg_sh.at[pl.ds(r0, ROWS_PER_TILE)])

        @pl.when(sid == NS - 1)
        def _():
            pltpu.sync_copy(u_hbm.at[pl.ds(TAIL_R0, TAIL_ROWS)],
                            agg_sh.at[pl.ds(TAIL_R0, TAIL_ROWS)])

    @pl.when(cid == 1)
    def _():
        pltpu.sync_copy(zeros_nf_hbm.at[pl.ds(r0, ROWS_PER_TILE)],
                        agg_sh.at[pl.ds(r0, ROWS_PER_TILE)])

        @pl.when(sid == NS - 1)
        def _():
            pltpu.sync_copy(zeros_nf_hbm.at[pl.ds(TAIL_R0, TAIL_ROWS)],
                            agg_sh.at[pl.ds(TAIL_R0, TAIL_ROWS)])

    pltpu.sync_copy(src3_hbm.at[wid], src_v)
    for b in range(NBUF):
        pltpu.async_copy(u_hbm.at[src_v.at[b]], rows[b], gsem[b])
    pltpu.sync_copy(dst3_hbm.at[wid], dst_v)
    plsc.subcore_barrier()

    def round_body(jj, carry):
        for b in range(NBUF):
            j = jj * NBUF + b
            pltpu.make_async_copy(u_hbm.at[src_v.at[j]], rows[b],
                                  gsem[b]).wait()
            pltpu.async_copy(rows[b], agg_sh.at[dst_v.at[j]], ssem[b],
                             add=True)
            jn = j + NBUF

            @pl.when(jn < ITERS)
            def _():
                pltpu.make_async_copy(rows[b], agg_sh.at[dst_v.at[j]],
                                      ssem[b]).wait()
                pltpu.async_copy(u_hbm.at[src_v.at[jn]], rows[b], gsem[b])
        return carry

    lax.fori_loop(0, ITERS // NBUF, round_body, 0)
    for b in range(ITERS % NBUF):
        j = (ITERS // NBUF) * NBUF + b
        pltpu.make_async_copy(u_hbm.at[src_v.at[j]], rows[b], gsem[b]).wait()
        pltpu.async_copy(rows[b], agg_sh.at[dst_v.at[j]], ssem[b], add=True)
    for t in range(NBUF):
        j = ITERS - NBUF + t
        pltpu.make_async_copy(rows[j % NBUF], agg_sh.at[dst_v.at[j]],
                              ssem[j % NBUF]).wait()
    plsc.subcore_barrier()

    pltpu.sync_copy(agg_sh.at[pl.ds(r0, ROWS_PER_TILE)],
                    out_hbm.at[cid, pl.ds(r0, ROWS_PER_TILE)])

    @pl.when(sid == NS - 1)
    def _():
        pltpu.sync_copy(agg_sh.at[pl.ds(TAIL_R0, TAIL_ROWS)],
                        out_hbm.at[cid, pl.ds(TAIL_R0, TAIL_ROWS)])


@functools.partial(
    pl.kernel,
    out_type=jax.ShapeDtypeStruct((NC, N), jnp.float32),
    mesh=_MESH,
    compiler_params=_SC_PARAMS,
    scratch_types=[
        pltpu.VMEM((ITERS, C), jnp.int32),
        pltpu.VMEM((ITERS, C), jnp.int32),
        pltpu.VMEM((ITERS, C), jnp.float32),
        pltpu.VMEM_SHARED((N,), jnp.float32),
        pltpu.VMEM_SHARED((N,), jnp.float32),
        pltpu.SemaphoreType.DMA,
        pltpu.SemaphoreType.DMA,
    ],
)
def _agg_scalar_kernel(src3_hbm, dst3_hbm, uv_hbm, zeros_n_hbm, out_hbm,
                       src_v, dst_v, vals_v, uv_sh, acc_sh, gsem, ssem):
    cid = lax.axis_index("c")
    sid = lax.axis_index("s")
    wid = _wid(cid, sid)

    @pl.when(sid == 0)
    def _():
        pltpu.sync_copy(zeros_n_hbm, acc_sh)
        pltpu.sync_copy(uv_hbm, uv_sh)

    pltpu.sync_copy(src3_hbm.at[wid], src_v)
    pltpu.sync_copy(dst3_hbm.at[wid], dst_v)
    plsc.subcore_barrier()

    def gfire(j, carry):
        pltpu.async_copy(uv_sh.at[src_v.at[j]], vals_v.at[j], gsem)
        return carry

    def gdrain_sfire(j, carry):
        pltpu.make_async_copy(uv_sh.at[src_v.at[j]], vals_v.at[j],
                              gsem).wait()
        pltpu.async_copy(vals_v.at[j], acc_sh.at[dst_v.at[j]], ssem,
                         add=True)
        return carry

    def sdrain(j, carry):
        pltpu.make_async_copy(vals_v.at[j], acc_sh.at[dst_v.at[j]],
                              ssem).wait()
        return carry

    lax.fori_loop(0, ITERS, gfire, 0)
    lax.fori_loop(0, ITERS, gdrain_sfire, 0)
    lax.fori_loop(0, ITERS, sdrain, 0)
    plsc.subcore_barrier()

    @pl.when(sid == 0)
    def _():
        pltpu.sync_copy(acc_sh, out_hbm.at[cid])


B = 1000
NB = N // B


def _scale_body(d0_ref, d1_ref, x_ref, w1_ref, dinv_ref, u_ref):
    deg = d0_ref[0, 0, :] + d1_ref[0, 0, :] + 1.0
    dinv = lax.rsqrt(deg)
    dinv_ref[0, 0, :] = dinv
    xw = jnp.dot(x_ref[...], w1_ref[...], preferred_element_type=jnp.float32)
    u_ref[...] = xw * dinv[:, None]


def _scale_call(d0, d1, x, W1):
    sspec = pl.BlockSpec((1, 1, B), lambda i: (i, 0, 0))
    fspec = pl.BlockSpec((B, F), lambda i: (i, 0))
    return pl.pallas_call(
        _scale_body,
        grid=(NB,),
        in_specs=[sspec, sspec, fspec,
                  pl.BlockSpec((F, H), lambda i: (0, 0))],
        out_specs=[sspec, pl.BlockSpec((B, H), lambda i: (i, 0))],
        out_shape=[
            jax.ShapeDtypeStruct((NB, 1, B), jnp.float32),
            jax.ShapeDtypeStruct((N, H), jnp.float32),
        ],
    )(d0.reshape(NB, 1, B), d1.reshape(NB, 1, B), x, W1)


def _dense_body(a0_ref, a1_ref, dinv_ref, b1_ref, w2_ref, uv_ref):
    dinv = dinv_ref[0, 0, :]
    s = a0_ref[...].astype(jnp.float32) + a1_ref[...].astype(jnp.float32)
    h = jnp.maximum(s * dinv[:, None] + b1_ref[...], 0.0)
    v = jnp.dot(h, w2_ref[...], preferred_element_type=jnp.float32)
    uv_ref[0, 0, :] = dinv * v[:, 0]


def _dense_call(a0, a1, dinv3, b1, W2):
    sspec = pl.BlockSpec((1, 1, B), lambda i: (i, 0, 0))
    fspec = pl.BlockSpec((B, H), lambda i: (i, 0))
    return pl.pallas_call(
        _dense_body,
        grid=(NB,),
        in_specs=[
            fspec, fspec, sspec,
            pl.BlockSpec((1, H), lambda i: (0, 0)),
            pl.BlockSpec((H, 1), lambda i: (0, 0)),
        ],
        out_specs=sspec,
        out_shape=jax.ShapeDtypeStruct((NB, 1, B), jnp.float32),
    )(a0, a1, dinv3, b1.reshape(1, H), W2)


def _final_body(p0_ref, p1_ref, uv_ref, dinv_ref, b2_ref, out_ref):
    out_ref[...] = (dinv_ref[...] * (p0_ref[...] + p1_ref[...] + uv_ref[...])
                    + b2_ref[0, 0])


def _final_call(p0, p1, uv3, dinv3, b2):
    sspec = pl.BlockSpec((1, 1, B), lambda i: (i, 0, 0))
    return pl.pallas_call(
        _final_body,
        grid=(NB,),
        in_specs=[sspec, sspec, sspec, sspec,
                  pl.BlockSpec((1, 1), lambda i: (0, 0))],
        out_specs=sspec,
        out_shape=jax.ShapeDtypeStruct((NB, 1, B), jnp.float32),
    )(p0.reshape(NB, 1, B), p1.reshape(NB, 1, B), uv3, dinv3,
      b2.reshape(1, 1))


def kernel(x, edge_index, W1, b1, W2, b2):
    src = edge_index[0].astype(jnp.int32)
    dst = edge_index[1].astype(jnp.int32)
    src3 = src.reshape(NW, ITERS, C)
    dst3 = dst.reshape(NW, ITERS, C)
    dstd = dst.reshape(NW, DEG_ITERS, DEG_C)
    zeros_n = jnp.zeros((N,), jnp.float32)
    zeros_nf = jnp.zeros((N, F), jnp.float32)

    degp = _deg_kernel(dstd, zeros_n)
    dinv3, u = _scale_call(degp[0], degp[1], x, W1)

    aggp = _agg_rows_kernel(src3, dst3, u, zeros_nf)
    uv3 = _dense_call(aggp[0], aggp[1], dinv3, b1, W2)

    accp = _agg_scalar_kernel(src3, dst3, uv3.reshape(N), zeros_n)
    out3 = _final_call(accp[0], accp[1], uv3, dinv3, b2)
    return out3.reshape(N)


# speedup vs baseline: 53.7458x; 1.0449x over previous
import functools

import jax
import jax.numpy as jnp
from jax import lax
from jax.experimental import pallas as pl
from jax.experimental.pallas import tpu as pltpu
from jax.experimental.pallas import tpu_sc as plsc

N = 10000
F = 128
H = 128
E = 320000

NC = 2
NS = 16
NW = NC * NS
LANES = 16

E_PER_W = E // NW
C = 80
ITERS = E_PER_W // C
NBUF = 3
WAVE = 20

ROWS_PER_TILE = 624
TAIL_ROWS = N - NS * ROWS_PER_TILE
TAIL_R0 = NS * ROWS_PER_TILE

_MESH = plsc.VectorSubcoreMesh(core_axis_name="c", subcore_axis_name="s",
                               num_cores=NC, num_subcores=NS)

_SC_PARAMS = pltpu.CompilerParams(use_tc_tiling_on_sc=False)


def _wid(cid, sid):
    return sid * NC + cid


DEG_C = 125
DEG_ITERS = E_PER_W // DEG_C
DEG_WAVES = DEG_ITERS // WAVE


@functools.partial(
    pl.kernel,
    out_type=jax.ShapeDtypeStruct((NC, N), jnp.float32),
    mesh=_MESH,
    compiler_params=_SC_PARAMS,
    scratch_types=[
        pltpu.VMEM((DEG_ITERS, DEG_C), jnp.int32),
        pltpu.VMEM((128,), jnp.float32),
        pltpu.VMEM_SHARED((N,), jnp.float32),
        pltpu.SemaphoreType.DMA,
    ],
)
def _deg_kernel(dstd_hbm, zeros_n_hbm, out_hbm, idx_v, ones_v, deg_sh, sem):
    cid = lax.axis_index("c")
    sid = lax.axis_index("s")
    wid = _wid(cid, sid)

    @pl.when(sid == 0)
    def _():
        pltpu.sync_copy(zeros_n_hbm, deg_sh)

    for k in range(128 // LANES):
        ones_v[pl.ds(k * LANES, LANES)] = jnp.ones((LANES,), jnp.float32)
    pltpu.sync_copy(dstd_hbm.at[wid], idx_v)
    plsc.subcore_barrier()

    ones_c = ones_v.at[pl.ds(0, DEG_C)]

    def wave_body(w, carry):
        for k in range(WAVE):
            j = w * WAVE + k
            pltpu.async_copy(ones_c, deg_sh.at[idx_v.at[j]], sem, add=True)
        for k in range(WAVE):
            pltpu.make_async_copy(
                ones_c, deg_sh.at[idx_v.at[w * WAVE + k]], sem).wait()
        return carry

    lax.fori_loop(0, DEG_WAVES, wave_body, 0)
    plsc.subcore_barrier()

    @pl.when(sid == 0)
    def _():
        pltpu.sync_copy(deg_sh, out_hbm.at[cid])


@functools.partial(
    pl.kernel,
    out_type=jax.ShapeDtypeStruct((NC, N, F), jnp.float32),
    mesh=_MESH,
    compiler_params=_SC_PARAMS,
    scratch_types=(
        [pltpu.VMEM((ITERS, C), jnp.int32),
         pltpu.VMEM((ITERS, C), jnp.int32),
         pltpu.VMEM_SHARED((N, F), jnp.float32)]
        + [pltpu.VMEM((C, F), jnp.float32) for _ in range(NBUF)]
        + [pltpu.SemaphoreType.DMA for _ in range(2 * NBUF)]
    ),
)
def _agg_rows_kernel(src3_hbm, dst3_hbm, u_hbm, zeros_nf_hbm, out_hbm,
                     src_v, dst_v, agg_sh, *bufs_and_sems):
    rows = bufs_and_sems[:NBUF]
    gsem = bufs_and_sems[NBUF:2 * NBUF]
    ssem = bufs_and_sems[2 * NBUF:]
    cid = lax.axis_index("c")
    sid = lax.axis_index("s")
    wid = _wid(cid, sid)

    r0 = pl.multiple_of(sid * ROWS_PER_TILE, 8)
    @pl.when(cid == 0)
    def _():
        pltpu.sync_copy(u_hbm.at[pl.ds(r0, ROWS_PER_TILE)],
                        agg_sh.at[pl.ds(r0, ROWS_PER_TILE)])

        @pl.when(sid == NS - 1)
        def _():
            pltpu.sync_copy(u_hbm.at[pl.ds(TAIL_R0, TAIL_ROWS)],
                            agg_sh.at[pl.ds(TAIL_R0, TAIL_ROWS)])

    @pl.when(cid == 1)
    def _():
        pltpu.sync_copy(zeros_nf_hbm.at[pl.ds(r0, ROWS_PER_TILE)],
                        agg_sh.at[pl.ds(r0, ROWS_PER_TILE)])

        @pl.when(sid == NS - 1)
        def _():
            pltpu.sync_copy(zeros_nf_hbm.at[pl.ds(TAIL_R0, TAIL_ROWS)],
                            agg_sh.at[pl.ds(TAIL_R0, TAIL_ROWS)])

    pltpu.sync_copy(src3_hbm.at[wid], src_v)
    for b in range(2):
        pltpu.async_copy(u_hbm.at[src_v.at[b]], rows[b], gsem[b])
    pltpu.sync_copy(dst3_hbm.at[wid], dst_v)
    plsc.subcore_barrier()

    def _step(j, b):
        bp = (b + 2) % NBUF
        pltpu.make_async_copy(u_hbm.at[src_v.at[j]], rows[b],
                              gsem[b]).wait()
        pltpu.async_copy(rows[b], agg_sh.at[dst_v.at[j]], ssem[b], add=True)

        @pl.when(j >= 1)
        def _():
            pltpu.make_async_copy(rows[bp], agg_sh.at[dst_v.at[j - 1]],
                                  ssem[bp]).wait()

        @pl.when(j + 2 < ITERS)
        def _():
            pltpu.async_copy(u_hbm.at[src_v.at[j + 2]], rows[bp], gsem[bp])

    def round_body(jj, carry):
        for b in range(NBUF):
            _step(jj * NBUF + b, b)
        return carry

    lax.fori_loop(0, ITERS // NBUF, round_body, 0)
    for t in range(ITERS % NBUF):
        j = (ITERS // NBUF) * NBUF + t
        _step(j, j % NBUF)
    pltpu.make_async_copy(rows[(ITERS - 1) % NBUF],
                          agg_sh.at[dst_v.at[ITERS - 1]],
                          ssem[(ITERS - 1) % NBUF]).wait()
    plsc.subcore_barrier()

    pltpu.sync_copy(agg_sh.at[pl.ds(r0, ROWS_PER_TILE)],
                    out_hbm.at[cid, pl.ds(r0, ROWS_PER_TILE)])

    @pl.when(sid == NS - 1)
    def _():
        pltpu.sync_copy(agg_sh.at[pl.ds(TAIL_R0, TAIL_ROWS)],
                        out_hbm.at[cid, pl.ds(TAIL_R0, TAIL_ROWS)])


@functools.partial(
    pl.kernel,
    out_type=jax.ShapeDtypeStruct((NC, N), jnp.float32),
    mesh=_MESH,
    compiler_params=_SC_PARAMS,
    scratch_types=[
        pltpu.VMEM((ITERS, C), jnp.int32),
        pltpu.VMEM((ITERS, C), jnp.int32),
        pltpu.VMEM((ITERS, C), jnp.float32),
        pltpu.VMEM_SHARED((N,), jnp.float32),
        pltpu.VMEM_SHARED((N,), jnp.float32),
        pltpu.SemaphoreType.DMA,
        pltpu.SemaphoreType.DMA,
    ],
)
def _agg_scalar_kernel(src3_hbm, dst3_hbm, uv_hbm, zeros_n_hbm, out_hbm,
                       src_v, dst_v, vals_v, uv_sh, acc_sh, gsem, ssem):
    cid = lax.axis_index("c")
    sid = lax.axis_index("s")
    wid = _wid(cid, sid)

    @pl.when(sid == 0)
    def _():
        pltpu.sync_copy(zeros_n_hbm, acc_sh)
        pltpu.sync_copy(uv_hbm, uv_sh)

    pltpu.sync_copy(src3_hbm.at[wid], src_v)
    pltpu.sync_copy(dst3_hbm.at[wid], dst_v)
    plsc.subcore_barrier()

    def gfire(j, carry):
        pltpu.async_copy(uv_sh.at[src_v.at[j]], vals_v.at[j], gsem)
        return carry

    def gdrain_sfire(j, carry):
        pltpu.make_async_copy(uv_sh.at[src_v.at[j]], vals_v.at[j],
                              gsem).wait()
        pltpu.async_copy(vals_v.at[j], acc_sh.at[dst_v.at[j]], ssem,
                         add=True)
        return carry

    def sdrain(j, carry):
        pltpu.make_async_copy(vals_v.at[j], acc_sh.at[dst_v.at[j]],
                              ssem).wait()
        return carry

    lax.fori_loop(0, ITERS, gfire, 0)
    lax.fori_loop(0, ITERS, gdrain_sfire, 0)
    lax.fori_loop(0, ITERS, sdrain, 0)
    plsc.subcore_barrier()

    @pl.when(sid == 0)
    def _():
        pltpu.sync_copy(acc_sh, out_hbm.at[cid])


B = 1000
NB = N // B


def _scale_body(d0_ref, d1_ref, x_ref, w1_ref, dinv_ref, u_ref):
    deg = d0_ref[0, 0, :] + d1_ref[0, 0, :] + 1.0
    dinv = lax.rsqrt(deg)
    dinv_ref[0, 0, :] = dinv
    xw = jnp.dot(x_ref[...], w1_ref[...], preferred_element_type=jnp.float32)
    u_ref[...] = xw * dinv[:, None]


def _scale_call(d0, d1, x, W1):
    sspec = pl.BlockSpec((1, 1, B), lambda i: (i, 0, 0))
    fspec = pl.BlockSpec((B, F), lambda i: (i, 0))
    return pl.pallas_call(
        _scale_body,
        grid=(NB,),
        in_specs=[sspec, sspec, fspec,
                  pl.BlockSpec((F, H), lambda i: (0, 0))],
        out_specs=[sspec, pl.BlockSpec((B, H), lambda i: (i, 0))],
        out_shape=[
            jax.ShapeDtypeStruct((NB, 1, B), jnp.float32),
            jax.ShapeDtypeStruct((N, H), jnp.float32),
        ],
    )(d0.reshape(NB, 1, B), d1.reshape(NB, 1, B), x, W1)


def _dense_body(a0_ref, a1_ref, dinv_ref, b1_ref, w2_ref, uv_ref):
    dinv = dinv_ref[0, 0, :]
    s = a0_ref[...].astype(jnp.float32) + a1_ref[...].astype(jnp.float32)
    h = jnp.maximum(s * dinv[:, None] + b1_ref[...], 0.0)
    v = jnp.dot(h, w2_ref[...], preferred_element_type=jnp.float32)
    uv_ref[0, 0, :] = dinv * v[:, 0]


def _dense_call(a0, a1, dinv3, b1, W2):
    sspec = pl.BlockSpec((1, 1, B), lambda i: (i, 0, 0))
    fspec = pl.BlockSpec((B, H), lambda i: (i, 0))
    return pl.pallas_call(
        _dense_body,
        grid=(NB,),
        in_specs=[
            fspec, fspec, sspec,
            pl.BlockSpec((1, H), lambda i: (0, 0)),
            pl.BlockSpec((H, 1), lambda i: (0, 0)),
        ],
        out_specs=sspec,
        out_shape=jax.ShapeDtypeStruct((NB, 1, B), jnp.float32),
    )(a0, a1, dinv3, b1.reshape(1, H), W2)


def _final_body(p0_ref, p1_ref, uv_ref, dinv_ref, b2_ref, out_ref):
    out_ref[...] = (dinv_ref[...] * (p0_ref[...] + p1_ref[...] + uv_ref[...])
                    + b2_ref[0, 0])


def _final_call(p0, p1, uv3, dinv3, b2):
    sspec = pl.BlockSpec((1, 1, B), lambda i: (i, 0, 0))
    return pl.pallas_call(
        _final_body,
        grid=(NB,),
        in_specs=[sspec, sspec, sspec, sspec,
                  pl.BlockSpec((1, 1), lambda i: (0, 0))],
        out_specs=sspec,
        out_shape=jax.ShapeDtypeStruct((NB, 1, B), jnp.float32),
    )(p0.reshape(NB, 1, B), p1.reshape(NB, 1, B), uv3, dinv3,
      b2.reshape(1, 1))


def kernel(x, edge_index, W1, b1, W2, b2):
    src = edge_index[0].astype(jnp.int32)
    dst = edge_index[1].astype(jnp.int32)
    src3 = src.reshape(NW, ITERS, C)
    dst3 = dst.reshape(NW, ITERS, C)
    dstd = dst.reshape(NW, DEG_ITERS, DEG_C)
    zeros_n = jnp.zeros((N,), jnp.float32)
    zeros_nf = jnp.zeros((N, F), jnp.float32)

    degp = _deg_kernel(dstd, zeros_n)
    dinv3, u = _scale_call(degp[0], degp[1], x, W1)

    aggp = _agg_rows_kernel(src3, dst3, u, zeros_nf)
    uv3 = _dense_call(aggp[0], aggp[1], dinv3, b1, W2)

    accp = _agg_scalar_kernel(src3, dst3, uv3.reshape(N), zeros_n)
    out3 = _final_call(accp[0], accp[1], uv3, dinv3, b2)
    return out3.reshape(N)
